# baseline (speedup 1.0000x reference)
import functools

import jax
import jax.numpy as jnp
from jax import lax
from jax.experimental import pallas as pl
from jax.experimental.pallas import tpu as pltpu
from jax.experimental.pallas import tpu_sc as plsc

N = 10000
E = 320000
D = 128
EPS = 1e-5

NC = 2
NS = 16
NW = NC * NS
CHUNK = 128
CHUNKS_PER_W = 80
E_PAD = NW * CHUNKS_PER_W * CHUNK
N_PAD = 10240
ROWS_PER_TILE = N_PAD // NS
STAGE = 128


def _sc_conv_body(x_hbm, srcg, dstg, ewg, out_hbm,
                  src_v, dst_v, ew_v, rows0,
                  gs0, ss0, acc):
    cid = lax.axis_index("c")
    sid = lax.axis_index("s")
    wid = cid * NS + sid

    def zero_row(i, carry):
        for c in range(D // 16):
            rows0[i, pl.ds(c * 16, 16)] = jnp.zeros((16,), jnp.float32)
        return carry
    lax.fori_loop(0, STAGE, zero_row, 0)
    for t in range(ROWS_PER_TILE // STAGE):
        pltpu.sync_copy(rows0, acc.at[pl.ds(sid * ROWS_PER_TILE + t * STAGE, STAGE)])

    pltpu.sync_copy(srcg.at[wid], src_v)
    pltpu.sync_copy(dstg.at[wid], dst_v)
    pltpu.sync_copy(ewg.at[wid], ew_v)
    plsc.subcore_barrier()

    def scale(j, buf):
        def group(g, c2):
            wv = ew_v[j, pl.ds(g * 16, 16)]
            for l in range(16):
                w = wv[l]
                i = g * 16 + l
                for c in range(D // 16):
                    sl = pl.ds(c * 16, 16)
                    buf[i, sl] = buf[i, sl] * w
            return c2
        lax.fori_loop(0, CHUNK // 16, group, 0)

    pltpu.async_copy(x_hbm.at[src_v.at[0]], rows0, gs0)

    def pipe(j, carry):
        pltpu.make_async_copy(x_hbm.at[src_v.at[j]], rows0, gs0).wait()
        scale(j, rows0)
        pltpu.async_copy(rows0, acc.at[dst_v.at[j]], ss0, add=True)
        pltpu.make_async_copy(rows0, acc.at[dst_v.at[j]], ss0).wait()
        pltpu.async_copy(x_hbm.at[src_v.at[j + 1]], rows0, gs0)
        return carry
    lax.fori_loop(0, CHUNKS_PER_W - 1, pipe, 0)
    j = CHUNKS_PER_W - 1
    pltpu.make_async_copy(x_hbm.at[src_v.at[j]], rows0, gs0).wait()
    scale(j, rows0)
    pltpu.sync_copy(rows0, acc.at[dst_v.at[j]], add=True)
    plsc.subcore_barrier()

    for t in range(ROWS_PER_TILE // STAGE):
        r0 = sid * ROWS_PER_TILE + t * STAGE
        pltpu.sync_copy(acc.at[pl.ds(r0, STAGE)], rows0)
        pltpu.sync_copy(rows0, out_hbm.at[cid, pl.ds(r0, STAGE)])


@functools.cache
def _sc_conv_kernel():
    return pl.kernel(
        _sc_conv_body,
        out_type=jax.ShapeDtypeStruct((NC, N_PAD, D), jnp.float32),
        mesh=plsc.VectorSubcoreMesh(core_axis_name="c", subcore_axis_name="s",
                                    num_cores=NC, num_subcores=NS),
        scratch_types=[
            pltpu.VMEM((CHUNKS_PER_W, CHUNK), jnp.int32),
            pltpu.VMEM((CHUNKS_PER_W, CHUNK), jnp.int32),
            pltpu.VMEM((CHUNKS_PER_W, CHUNK), jnp.float32),
            pltpu.VMEM((CHUNK, D), jnp.float32),
            pltpu.SemaphoreType.DMA,
            pltpu.SemaphoreType.DMA,
            pltpu.VMEM_SHARED((N_PAD, D), jnp.float32),
        ],
    )


def _sc_conv(x, src, dst, ew):
    return _sc_conv_kernel()(x, src, dst, ew)


def _tc_post_body(p_ref, x_ref, wrel_ref, wroot_ref, b_ref, g_ref, be_ref,
                  o_ref, *, sig):
    agg = p_ref[0, :N, :] + p_ref[1, :N, :]
    h = lax.dot_general(agg, wrel_ref[...], (((1,), (1,)), ((), ())),
                        preferred_element_type=jnp.float32)
    h = h + lax.dot_general(x_ref[...], wroot_ref[...], (((1,), (1,)), ((), ())),
                            preferred_element_type=jnp.float32)
    h = h + b_ref[...]
    h = jnp.maximum(h, 0.0)
    mu = jnp.mean(h, axis=0, keepdims=True)
    var = jnp.mean((h - mu) * (h - mu), axis=0, keepdims=True)
    y = (h - mu) * lax.rsqrt(var + EPS) * g_ref[...] + be_ref[...]
    if sig:
        y = jax.nn.sigmoid(y)
    o_ref[...] = y


def _tc_post(p, x, wrel, wroot, b, gamma, beta, sig):
    return pl.pallas_call(
        functools.partial(_tc_post_body, sig=sig),
        out_shape=jax.ShapeDtypeStruct((N, D), jnp.float32),
    )(p, x, wrel, wroot, b.reshape(1, D), gamma.reshape(1, D),
      beta.reshape(1, D))


def kernel(x, edge_index, edge_attr, W_rel0, W_root0, b0, gamma0, beta0,
           W_rel1, W_root1, b1, gamma1, beta1):
    pad = E_PAD - E
    src = jnp.pad(edge_index[0], (0, pad)).reshape(NW, CHUNKS_PER_W, CHUNK)
    dst = jnp.pad(edge_index[1], (0, pad)).reshape(NW, CHUNKS_PER_W, CHUNK)
    ew = jnp.pad(edge_attr, (0, pad)).reshape(NW, CHUNKS_PER_W, CHUNK)

    p0 = _sc_conv(x, src, dst, ew)
    h1 = _tc_post(p0, x, W_rel0, W_root0, b0, gamma0, beta0, False)
    p1 = _sc_conv(h1, src, dst, ew)
    return _tc_post(p1, h1, W_rel1, W_root1, b1, gamma1, beta1, True)


# speedup vs baseline: 1.2249x; 1.2249x over previous
import functools

import jax
import jax.numpy as jnp
from jax import lax
from jax.experimental import pallas as pl
from jax.experimental.pallas import tpu as pltpu
from jax.experimental.pallas import tpu_sc as plsc

N = 10000
E = 320000
D = 128
EPS = 1e-5

NC = 2
NS = 16
NW = NC * NS
CHUNK = 64
CHUNKS_PER_W = 160
E_PAD = NW * CHUNKS_PER_W * CHUNK
N_PAD = 10240
ROWS_PER_TILE = N_PAD // NS
STAGE = 64


def _sc_conv_body(x_hbm, srcg, dstg, ewg, out_hbm,
                  src_v, dst_v, ew_v, rows0, rows1,
                  gs0, gs1, ss0, ss1, acc):
    cid = lax.axis_index("c")
    sid = lax.axis_index("s")
    wid = cid * NS + sid
    rows = (rows0, rows1)
    gsem = (gs0, gs1)
    ssem = (ss0, ss1)

    def zero_row(i, carry):
        for c in range(D // 16):
            rows0[i, pl.ds(c * 16, 16)] = jnp.zeros((16,), jnp.float32)
        return carry
    lax.fori_loop(0, STAGE, zero_row, 0)
    for t in range(ROWS_PER_TILE // STAGE):
        pltpu.sync_copy(rows0, acc.at[pl.ds(sid * ROWS_PER_TILE + t * STAGE, STAGE)])

    pltpu.sync_copy(srcg.at[wid], src_v)
    pltpu.sync_copy(dstg.at[wid], dst_v)
    pltpu.sync_copy(ewg.at[wid], ew_v)
    plsc.subcore_barrier()

    def idx(r, j):
        return r.at[lax.div(j, 2), pl.ds(lax.rem(j, 2) * CHUNK, CHUNK)]

    def scale(j, buf):
        def group(g, c2):
            wv = ew_v[lax.div(j, 2), pl.ds(lax.rem(j, 2) * CHUNK + g * 16, 16)]
            for l in range(16):
                w = wv[l]
                i = g * 16 + l
                for c in range(D // 16):
                    sl = pl.ds(c * 16, 16)
                    buf[i, sl] = buf[i, sl] * w
            return c2
        lax.fori_loop(0, CHUNK // 16, group, 0)

    pltpu.async_copy(x_hbm.at[idx(src_v, 0)], rows0, gs0)
    pltpu.async_copy(x_hbm.at[idx(src_v, 1)], rows1, gs1)

    def pipe(j2, carry):
        for b in range(2):
            j = j2 * 2 + b
            pltpu.make_async_copy(x_hbm.at[idx(src_v, j)], rows[b], gsem[b]).wait()
            scale(j, rows[b])
            pltpu.async_copy(rows[b], acc.at[idx(dst_v, j)], ssem[b], add=True)
            pltpu.make_async_copy(rows[b], acc.at[idx(dst_v, j)], ssem[b]).wait()
            pltpu.async_copy(x_hbm.at[idx(src_v, j + 2)], rows[b], gsem[b])
        return carry
    lax.fori_loop(0, CHUNKS_PER_W // 2 - 1, pipe, 0)
    for b in range(2):
        j = CHUNKS_PER_W - 2 + b
        pltpu.make_async_copy(x_hbm.at[idx(src_v, j)], rows[b], gsem[b]).wait()
        scale(j, rows[b])
        pltpu.sync_copy(rows[b], acc.at[idx(dst_v, j)], add=True)
    plsc.subcore_barrier()

    for t in range(ROWS_PER_TILE // STAGE):
        r0 = sid * ROWS_PER_TILE + t * STAGE
        pltpu.sync_copy(acc.at[pl.ds(r0, STAGE)], rows0)
        pltpu.sync_copy(rows0, out_hbm.at[cid, pl.ds(r0, STAGE)])


@functools.cache
def _sc_conv_kernel():
    return pl.kernel(
        _sc_conv_body,
        out_type=jax.ShapeDtypeStruct((NC, N_PAD, D), jnp.float32),
        mesh=plsc.VectorSubcoreMesh(core_axis_name="c", subcore_axis_name="s",
                                    num_cores=NC, num_subcores=NS),
        scratch_types=[
            pltpu.VMEM((CHUNKS_PER_W // 2, 2 * CHUNK), jnp.int32),
            pltpu.VMEM((CHUNKS_PER_W // 2, 2 * CHUNK), jnp.int32),
            pltpu.VMEM((CHUNKS_PER_W // 2, 2 * CHUNK), jnp.float32),
            pltpu.VMEM((CHUNK, D), jnp.float32),
            pltpu.VMEM((CHUNK, D), jnp.float32),
            pltpu.SemaphoreType.DMA,
            pltpu.SemaphoreType.DMA,
            pltpu.SemaphoreType.DMA,
            pltpu.SemaphoreType.DMA,
            pltpu.VMEM_SHARED((N_PAD, D), jnp.float32),
        ],
    )


def _sc_conv(x, src, dst, ew):
    return _sc_conv_kernel()(x, src, dst, ew)


def _tc_post_body(p_ref, x_ref, wrel_ref, wroot_ref, b_ref, g_ref, be_ref,
                  o_ref, *, sig):
    agg = p_ref[0, :N, :] + p_ref[1, :N, :]
    h = lax.dot_general(agg, wrel_ref[...], (((1,), (1,)), ((), ())),
                        preferred_element_type=jnp.float32)
    h = h + lax.dot_general(x_ref[...], wroot_ref[...], (((1,), (1,)), ((), ())),
                            preferred_element_type=jnp.float32)
    h = h + b_ref[...]
    h = jnp.maximum(h, 0.0)
    mu = jnp.mean(h, axis=0, keepdims=True)
    var = jnp.mean((h - mu) * (h - mu), axis=0, keepdims=True)
    y = (h - mu) * lax.rsqrt(var + EPS) * g_ref[...] + be_ref[...]
    if sig:
        y = jax.nn.sigmoid(y)
    o_ref[...] = y


def _tc_post(p, x, wrel, wroot, b, gamma, beta, sig):
    return pl.pallas_call(
        functools.partial(_tc_post_body, sig=sig),
        out_shape=jax.ShapeDtypeStruct((N, D), jnp.float32),
    )(p, x, wrel, wroot, b.reshape(1, D), gamma.reshape(1, D),
      beta.reshape(1, D))


def kernel(x, edge_index, edge_attr, W_rel0, W_root0, b0, gamma0, beta0,
           W_rel1, W_root1, b1, gamma1, beta1):
    pad = E_PAD - E
    shp = (NW, CHUNKS_PER_W // 2, 2 * CHUNK)
    src = jnp.pad(edge_index[0], (0, pad)).reshape(shp)
    dst = jnp.pad(edge_index[1], (0, pad)).reshape(shp)
    ew = jnp.pad(edge_attr, (0, pad)).reshape(shp)

    p0 = _sc_conv(x, src, dst, ew)
    h1 = _tc_post(p0, x, W_rel0, W_root0, b0, gamma0, beta0, False)
    p1 = _sc_conv(h1, src, dst, ew)
    return _tc_post(p1, h1, W_rel1, W_root1, b1, gamma1, beta1, True)
